# Initial kernel scaffold; baseline (speedup 1.0000x reference)
import jax
import jax.numpy as jnp
from jax.experimental import pallas as pl


def kernel(top_c, output):
    raise NotImplementedError("write your pallas kernel here")


# speedup vs baseline: 3.6679x; 3.6679x over previous
import jax
import jax.numpy as jnp
from jax.experimental import pallas as pl
from jax.experimental.pallas import tpu as pltpu

_B = 128
_N = 100000
_K = 64
_R = 8
_ONE_BITS = 0x3F800000


def _loss_kernel(x_ref, out_ref):
    x = x_ref[...]
    bits = jax.lax.bitcast_convert_type(x, jnp.int32)

    lo0 = jnp.zeros((_R, 1), jnp.int32)
    hi0 = jnp.full((_R, 1), _ONE_BITS, jnp.int32)

    def body(_, carry):
        lo, hi = carry
        mid = (lo + hi) // 2
        cnt = jnp.sum((bits >= mid).astype(jnp.int32), axis=1, keepdims=True)
        take = cnt >= _K
        return jnp.where(take, mid, lo), jnp.where(take, hi, mid)

    lo, _ = jax.lax.fori_loop(0, 30, body, (lo0, hi0))
    t = jax.lax.bitcast_convert_type(lo, jnp.float32)

    l = jnp.log(1.0 - x)
    s_all = jnp.sum(l)
    gt = x > t
    cnt_gt = jnp.sum(gt.astype(jnp.float32), axis=1, keepdims=True)
    sum_gt = jnp.sum(jnp.where(gt, l, 0.0))
    t_term = jnp.sum((jnp.float32(_K) - cnt_gt) * jnp.log(1.0 - t))
    partial = s_all - (sum_gt + t_term)

    @pl.when(pl.program_id(0) == 0)
    def _():
        out_ref[0, 0] = 0.0

    out_ref[0, 0] += -partial / jnp.float32(_N)


def kernel(top_c, output):
    out = pl.pallas_call(
        _loss_kernel,
        grid=(_B // _R,),
        in_specs=[pl.BlockSpec((_R, _N), lambda i: (i, 0))],
        out_specs=pl.BlockSpec(
            (1, 1), lambda i: (0, 0), memory_space=pltpu.SMEM
        ),
        out_shape=jax.ShapeDtypeStruct((1, 1), jnp.float32),
    )(output)
    loss = out[0, 0]
    return loss + 0.0 * jnp.asarray(top_c, dtype=loss.dtype)


# speedup vs baseline: 6.4930x; 1.7703x over previous
import jax
import jax.numpy as jnp
from jax.experimental import pallas as pl
from jax.experimental.pallas import tpu as pltpu

_B = 128
_N = 100000
_K = 64
_R = 8
_S = 16
_G = _N // _S
_ONE_BITS = 0x3F800000


def _loss_kernel(x_ref, out_ref):
    x = x_ref[...]
    bits = jax.lax.bitcast_convert_type(x, jnp.int32)

    m = jnp.max(x, axis=1)
    mbits = jax.lax.bitcast_convert_type(m, jnp.int32)

    lo0 = jnp.zeros((_R, 1), jnp.int32)
    hi0 = jnp.full((_R, 1), _ONE_BITS, jnp.int32)

    def bis(_, carry):
        lo, hi = carry
        mid = (lo + hi) // 2
        cnt = jnp.sum((mbits >= mid).astype(jnp.int32), axis=1, keepdims=True)
        take = cnt >= _K
        return jnp.where(take, mid, lo), jnp.where(take, hi, mid)

    glo, _ = jax.lax.fori_loop(0, 30, bis, (lo0, hi0))
    gstar = glo[:, :, None]

    l = jnp.log(1.0 - x)
    s_all = jnp.sum(l)
    cand = bits >= gstar
    c0 = jnp.sum(cand.astype(jnp.int32), axis=(1, 2))[:, None, None]
    sum_cand = jnp.sum(jnp.where(cand, l, 0.0), axis=(1, 2))

    def cond(carry):
        _, c_rem, _ = carry
        return jnp.any(c_rem > _K)

    def body(carry):
        b, c_rem, acc = carry
        active = c_rem > _K
        inc = bits >= b
        mn = jnp.min(jnp.where(inc, x, 2.0), axis=(1, 2))[:, None, None]
        n_eq = jnp.sum((inc & (x == mn)).astype(jnp.int32),
                       axis=(1, 2))[:, None, None]
        rem_all = active & (c_rem - n_eq >= _K)
        rem_part = active & ~rem_all
        lm = jnp.log(1.0 - jnp.where(active, mn, 0.0))
        acc = acc + jnp.where(
            rem_all, n_eq.astype(jnp.float32) * lm,
            jnp.where(rem_part, (c_rem - _K).astype(jnp.float32) * lm, 0.0))
        c_rem = jnp.where(rem_all, c_rem - n_eq,
                          jnp.where(rem_part, _K, c_rem))
        mn_b = jax.lax.bitcast_convert_type(mn, jnp.int32)
        b = jnp.where(rem_all, mn_b + 1, b)
        return b, c_rem, acc

    _, _, acc = jax.lax.while_loop(
        cond, body, (gstar, c0, jnp.zeros((_R, 1, 1), jnp.float32)))

    t_sum = jnp.sum(sum_cand) - jnp.sum(acc)
    partial = s_all - t_sum

    @pl.when(pl.program_id(0) == 0)
    def _():
        out_ref[0, 0] = 0.0

    out_ref[0, 0] += -partial / jnp.float32(_N)


def kernel(top_c, output):
    x3 = output.reshape(_B, _S, _G)
    out = pl.pallas_call(
        _loss_kernel,
        grid=(_B // _R,),
        in_specs=[pl.BlockSpec((_R, _S, _G), lambda i: (i, 0, 0))],
        out_specs=pl.BlockSpec(
            (1, 1), lambda i: (0, 0), memory_space=pltpu.SMEM
        ),
        out_shape=jax.ShapeDtypeStruct((1, 1), jnp.float32),
    )(x3)
    loss = out[0, 0]
    return loss + 0.0 * jnp.asarray(top_c, dtype=loss.dtype)


# speedup vs baseline: 7.5631x; 1.1648x over previous
import jax
import jax.numpy as jnp
from jax.experimental import pallas as pl
from jax.experimental.pallas import tpu as pltpu

_B = 128
_N = 100000
_K = 64
_R = 16
_S = 32
_G = _N // _S
_ONE_BITS = 0x3F800000


def _loss_kernel(x_ref, out_ref):
    x = x_ref[...]
    bits = jax.lax.bitcast_convert_type(x, jnp.int32)

    l = jnp.log(1.0 - x)
    s_all = jnp.sum(l)
    m = jnp.max(x, axis=1)
    mbits = jax.lax.bitcast_convert_type(m, jnp.int32)

    lo0 = jnp.zeros((_R, 1), jnp.int32)
    hi0 = jnp.full((_R, 1), _ONE_BITS, jnp.int32)

    def bis(_, carry):
        lo, hi = carry
        mid = (lo + hi) // 2
        cnt = jnp.sum((mbits >= mid).astype(jnp.int32), axis=1, keepdims=True)
        take = cnt >= _K
        return jnp.where(take, mid, lo), jnp.where(take, hi, mid)

    glo, _ = jax.lax.fori_loop(0, 30, bis, (lo0, hi0))
    gstar = glo[:, :, None]

    cand = bits >= gstar
    c0 = jnp.sum(cand.astype(jnp.int32), axis=(1, 2))[:, None, None]
    sum_cand = jnp.sum(jnp.where(cand, l, 0.0), axis=(1, 2))

    def cond(carry):
        _, c_rem, _ = carry
        return jnp.any(c_rem > _K)

    def body(carry):
        b, c_rem, acc = carry
        active = c_rem > _K
        inc = bits >= b
        mn = jnp.min(jnp.where(inc, x, 2.0), axis=(1, 2))[:, None, None]
        n_eq = jnp.sum((inc & (x == mn)).astype(jnp.int32),
                       axis=(1, 2))[:, None, None]
        rem_all = active & (c_rem - n_eq >= _K)
        rem_part = active & ~rem_all
        lm = jnp.log(1.0 - jnp.where(active, mn, 0.0))
        acc = acc + jnp.where(
            rem_all, n_eq.astype(jnp.float32) * lm,
            jnp.where(rem_part, (c_rem - _K).astype(jnp.float32) * lm, 0.0))
        c_rem = jnp.where(rem_all, c_rem - n_eq,
                          jnp.where(rem_part, _K, c_rem))
        mn_b = jax.lax.bitcast_convert_type(mn, jnp.int32)
        b = jnp.where(rem_all, mn_b + 1, b)
        return b, c_rem, acc

    _, _, acc = jax.lax.while_loop(
        cond, body, (gstar, c0, jnp.zeros((_R, 1, 1), jnp.float32)))

    t_sum = jnp.sum(sum_cand) - jnp.sum(acc)
    partial = s_all - t_sum

    @pl.when(pl.program_id(0) == 0)
    def _():
        out_ref[0, 0] = 0.0

    out_ref[0, 0] += -partial / jnp.float32(_N)


def kernel(top_c, output):
    x3 = output.reshape(_B, _S, _G)
    out = pl.pallas_call(
        _loss_kernel,
        grid=(_B // _R,),
        in_specs=[pl.BlockSpec((_R, _S, _G), lambda i: (i, 0, 0))],
        out_specs=pl.BlockSpec(
            (1, 1), lambda i: (0, 0), memory_space=pltpu.SMEM
        ),
        out_shape=jax.ShapeDtypeStruct((1, 1), jnp.float32),
    )(x3)
    loss = out[0, 0]
    return loss + 0.0 * jnp.asarray(top_c, dtype=loss.dtype)
